# baseline (speedup 1.0000x reference)
import numpy as np
import jax
import jax.numpy as jnp
from jax import lax
from jax.experimental import pallas as pl
from jax.experimental.pallas import tpu as pltpu
from jax.experimental.pallas import tpu_sc as plsc

_C, _H, _W = 3, 32, 32
_B, _D = 64, 256
_P = _C * _H * _W
_L = _P
_R = _L * _B


def _zigzag_perm() -> np.ndarray:
    diagonals = [[] for _ in range(_H + _W - 1)]
    for i in range(_H):
        for j in range(_W):
            s = i + j
            if s % 2 == 0:
                diagonals[s].insert(0, (i, j))
            else:
                diagonals[s].append((i, j))
    pos = []
    for d in diagonals:
        for (i, j) in d:
            for c in range(_C):
                pos.append(c * _H * _W + i * _W + j)
    return np.asarray(pos, dtype=np.int32)


_PERM = _zigzag_perm()

_L_PER_W = 80
_L_TEC = 32 * _L_PER_W
_N_SCS = (_L - _L_TEC) // 2
_CHUNKS = _L_PER_W // 2
_HALF = _CHUNKS // 2
_HALF_SCS = _N_SCS // 2


def _sc_reorder(x3, perm):
    vmesh = plsc.VectorSubcoreMesh(core_axis_name="c", subcore_axis_name="s")
    smesh = plsc.ScalarSubcoreMesh(axis_name="c")

    def tec_fn(x3_hbm, perm_hbm, out_hbm,
               perm_v, b0, b1, g0, g1, s0, s1,
               perm_s, sp0, sp1, sg0, sg1, ss0, ss1):
        del perm_s, sp0, sp1, sg0, sg1, ss0, ss1
        bufs = (b0, b1)
        gsem = (g0, g1)
        ssem = (s0, s1)
        wid = lax.axis_index("s") * 2 + lax.axis_index("c")
        lbase = wid * _L_PER_W
        rbase = lbase * _B
        pltpu.sync_copy(
            perm_hbm.at[pl.ds(lbase, _L_PER_W)],
            perm_v.at[pl.ds(0, _L_PER_W)],
        )

        def gather_pair(chunk, b):
            v = perm_v[pl.ds(2 * chunk, 16)]
            p0 = v[0]
            p1 = v[1]
            return (
                pltpu.make_async_copy(
                    x3_hbm.at[:, p0, :], bufs[b].at[pl.ds(0, _B)], gsem[b]
                ),
                pltpu.make_async_copy(
                    x3_hbm.at[:, p1, :], bufs[b].at[pl.ds(_B, _B)], gsem[b]
                ),
            )

        def gather_start(chunk, b):
            ca, cb = gather_pair(chunk, b)
            ca.start()
            cb.start()

        def gather_wait(chunk, b):
            ca, cb = gather_pair(chunk, b)
            ca.wait()
            cb.wait()

        def scatter(chunk, b):
            return pltpu.make_async_copy(
                bufs[b], out_hbm.at[pl.ds(rbase + chunk * 2 * _B, 2 * _B)],
                ssem[b],
            )

        gather_start(0, 0)

        def body(t, carry):
            c0 = 2 * t
            c1 = c0 + 1

            @pl.when(t > 0)
            def _():
                scatter(c0 - 1, 1).wait()

            gather_start(c1, 1)
            gather_wait(c0, 0)
            scatter(c0, 0).start()
            gather_wait(c1, 1)
            scatter(c0, 0).wait()

            @pl.when(t < _HALF - 1)
            def _():
                gather_start(c1 + 1, 0)

            scatter(c1, 1).start()
            return carry

        lax.fori_loop(0, _HALF, body, 0)
        scatter(_CHUNKS - 1, 1).wait()

    def scs_fn(x3_hbm, perm_hbm, out_hbm,
               perm_v, b0, b1, g0, g1, s0, s1,
               perm_s, sp0, sp1, sg0, sg1, ss0, ss1):
        del perm_v, b0, b1, g0, g1, s0, s1
        bufs = (sp0, sp1)
        gsem = (sg0, sg1)
        ssem = (ss0, ss1)
        cid = lax.axis_index("c")
        l0 = _L_TEC + cid * _N_SCS
        pltpu.sync_copy(perm_hbm.at[pl.ds(l0, _N_SCS)], perm_s)

        def gather(i, k):
            p = perm_s[i]
            return pltpu.make_async_copy(x3_hbm.at[:, p, :], bufs[k], gsem[k])

        def scatter(i, k):
            return pltpu.make_async_copy(
                bufs[k], out_hbm.at[pl.ds((l0 + i) * _B, _B)], ssem[k]
            )

        gather(0, 0).start()

        def body(t, carry):
            i0 = 2 * t
            i1 = i0 + 1

            @pl.when(t > 0)
            def _():
                scatter(i0 - 1, 1).wait()

            gather(i1, 1).start()
            gather(i0, 0).wait()
            scatter(i0, 0).start()
            gather(i1, 1).wait()
            scatter(i0, 0).wait()

            @pl.when(t < _HALF_SCS - 1)
            def _():
                gather(i1 + 1, 0).start()

            scatter(i1, 1).start()
            return carry

        lax.fori_loop(0, _HALF_SCS, body, 0)
        scatter(_N_SCS - 1, 1).wait()

    vvmem = pltpu.VMEM @ vmesh
    vsem = pltpu.SemaphoreType.DMA @ vmesh
    ssmem = pltpu.SMEM @ smesh
    scsem = pltpu.SemaphoreType.DMA @ smesh

    run = pl.kernel(
        [tec_fn, scs_fn],
        out_type=jax.ShapeDtypeStruct((_R, _D), jnp.float32),
        mesh=[vmesh, smesh],
        scratch_types=[
            vvmem((_L_PER_W + 16,), jnp.int32),
            vvmem((2 * _B, _D), jnp.float32),
            vvmem((2 * _B, _D), jnp.float32),
            vsem, vsem, vsem, vsem,
            ssmem((_N_SCS,), jnp.int32),
            pltpu.VMEM_SHARED((_B, _D), jnp.float32),
            pltpu.VMEM_SHARED((_B, _D), jnp.float32),
            scsem, scsem, scsem, scsem,
        ],
    )
    return run(x3, perm)


def kernel(x):
    x3 = x.reshape(_B, _P, _D)
    out = _sc_reorder(x3, jnp.asarray(_PERM))
    return out.reshape(_L, _B, _D)


# speedup vs baseline: 1.7845x; 1.7845x over previous
import functools

import numpy as np
import jax
import jax.numpy as jnp
from jax import lax
from jax.experimental import pallas as pl
from jax.experimental.pallas import tpu as pltpu
from jax.experimental.pallas import tpu_sc as plsc

_C, _H, _W = 3, 32, 32
_B, _D = 64, 256
_P = _C * _H * _W
_L = _P
_R = _L * _B


def _zigzag_gather_idx() -> np.ndarray:
    diagonals = [[] for _ in range(_H + _W - 1)]
    for i in range(_H):
        for j in range(_W):
            s = i + j
            if s % 2 == 0:
                diagonals[s].insert(0, (i, j))
            else:
                diagonals[s].append((i, j))
    pos = []
    for d in diagonals:
        for (i, j) in d:
            for c in range(_C):
                pos.append(c * _H * _W + i * _W + j)
    perm = np.asarray(pos, dtype=np.int64)
    r = np.arange(_R, dtype=np.int64)
    return ((r % _B) * _P + perm[r // _B]).astype(np.int32)


_GATHER_IDX = _zigzag_gather_idx()

_NW = 32
_ROWS_PER_W = _R // _NW
_K = 128
_CHUNKS = _ROWS_PER_W // _K


def _sc_gather(table, idx):
    mesh = plsc.VectorSubcoreMesh(core_axis_name="c", subcore_axis_name="s")

    @functools.partial(
        pl.kernel,
        mesh=mesh,
        out_type=jax.ShapeDtypeStruct((_R, _D), jnp.float32),
        scratch_types=[
            pltpu.VMEM((_ROWS_PER_W,), jnp.int32),
            pltpu.VMEM((_K, _D), jnp.float32),
            pltpu.VMEM((_K, _D), jnp.float32),
            pltpu.VMEM((_K, _D), jnp.float32),
            pltpu.SemaphoreType.DMA,
            pltpu.SemaphoreType.DMA,
            pltpu.SemaphoreType.DMA,
            pltpu.SemaphoreType.DMA,
            pltpu.SemaphoreType.DMA,
            pltpu.SemaphoreType.DMA,
        ],
    )
    def k(table_hbm, idx_hbm, out_hbm, idx_v, b0, b1, b2,
          g0, g1, g2, s0, s1, s2):
        bufs = (b0, b1, b2)
        gsem = (g0, g1, g2)
        ssem = (s0, s1, s2)
        wid = lax.axis_index("s") * 2 + lax.axis_index("c")
        base = wid * _ROWS_PER_W
        pltpu.sync_copy(idx_hbm.at[pl.ds(base, _ROWS_PER_W)], idx_v)

        def gather(chunk, b):
            return pltpu.make_async_copy(
                table_hbm.at[idx_v.at[pl.ds(chunk * _K, _K)]], bufs[b], gsem[b]
            )

        def scatter(chunk, b):
            return pltpu.make_async_copy(
                bufs[b], out_hbm.at[pl.ds(base + chunk * _K, _K)], ssem[b]
            )

        gather(0, 0).start()
        gather(1, 1).start()
        third = _CHUNKS // 3

        def body(t, carry):
            i0 = 3 * t
            for b in range(3):
                i = i0 + b
                bn = (b + 2) % 3
                gather(i, b).wait()
                scatter(i, b).start()

                if b == 0:
                    @pl.when(t > 0)
                    def _():
                        scatter(i - 1, bn).wait()

                    gather(i + 2, bn).start()
                else:
                    scatter(i - 1, bn).wait()

                    @pl.when(t < third - 1)
                    def _():
                        gather(i + 2, bn).start()
            return carry

        lax.fori_loop(0, third, body, 0)
        scatter(_CHUNKS - 1, 2).wait()

    return k(table, idx)


def kernel(x):
    table = x.reshape(_B * _P, _D)
    out = _sc_gather(table, jnp.asarray(_GATHER_IDX))
    return out.reshape(_L, _B, _D)
